# baseline (speedup 1.0000x reference)
import functools

import jax
import jax.numpy as jnp
from jax import lax
from jax.experimental import pallas as pl
from jax.experimental.pallas import tpu as pltpu
from jax.experimental.pallas import tpu_sc as plsc

_LANES = 16
_LEAD = 1


def _t2_body(emb_ref, w2_ref, b_ref, out_ref):
    out_ref[...] = (
        jnp.dot(emb_ref[...], w2_ref[...], preferred_element_type=jnp.float32)
        + b_ref[...]
    )


def _lane_bcast(v, j):
    idx = jnp.full((_LANES,), j, dtype=jnp.int32)
    return jnp.take_along_axis(v, idx, axis=0, mode="promise_in_bounds")


def _tanh_body(x_ref, out_ref):
    n_grp = x_ref.shape[1]
    for i in range(n_grp):
        t = jnp.tanh(x_ref[:, i, :])
        tt = jnp.transpose(t)
        for q in range(4):
            st = tt[32 * q:32 * (q + 1), :]
            out_ref[4 * i + q, :, 0, :, :] = st.reshape(4, 8, 128)


def kernel(lngs, lats, grid_id, emb_table, W, b):
    B, L = lngs.shape
    V, D = emb_table.shape
    N = B * L

    t2 = pl.pallas_call(
        _t2_body,
        out_shape=jax.ShapeDtypeStruct((V, D), jnp.float32),
    )(emb_table, W[2:], b.reshape(1, D))

    info = plsc.get_sparse_core_info()
    NW = info.num_cores * info.num_subcores
    CHUNK = 128
    per_w = N // NW
    n_chunks = per_w // CHUNK
    assert per_w % CHUNK == 0 and N % NW == 0

    gid = grid_id.reshape(NW, n_chunks, CHUNK).astype(jnp.int32)
    ln = lngs.reshape(NW, n_chunks, CHUNK)
    la = lats.reshape(NW, n_chunks, CHUNK)
    w01 = W[:2]

    mesh = plsc.VectorSubcoreMesh(core_axis_name="c", subcore_axis_name="s")

    @functools.partial(
        pl.kernel,
        out_type=jax.ShapeDtypeStruct((N * D // 128, 128), jnp.float32),
        mesh=mesh,
        compiler_params=pltpu.CompilerParams(
            use_tc_tiling_on_sc=False, skip_device_barrier=True),
        scratch_types=[
            pltpu.VMEM((n_chunks + _LEAD, CHUNK), jnp.int32),
            pltpu.VMEM((n_chunks, CHUNK), jnp.float32),
            pltpu.VMEM((n_chunks, CHUNK), jnp.float32),
            pltpu.VMEM((2, CHUNK, D), jnp.float32),
            pltpu.VMEM((2, CHUNK * D // 128, 128), jnp.float32),
            pltpu.VMEM((2, D), jnp.float32),
            pltpu.SemaphoreType.DMA,
            pltpu.SemaphoreType.DMA,
        ],
    )
    def sc_k(t2_hbm, gid_hbm, ln_hbm, la_hbm, w01_hbm, out_hbm,
             idx_v, ln_v, la_v, rin, rout, w_v, gsem, osem):
        wid = lax.axis_index("s") * info.num_cores + lax.axis_index("c")
        base = wid * per_w
        pltpu.sync_copy(gid_hbm.at[wid], idx_v.at[pl.ds(0, n_chunks)])
        pltpu.sync_copy(ln_hbm.at[wid], ln_v)
        pltpu.sync_copy(la_hbm.at[wid], la_v)
        pltpu.sync_copy(w01_hbm, w_v)
        zero16 = jnp.zeros((_LANES,), jnp.int32)
        for r in range(_LEAD):
            for k in range(CHUNK // _LANES):
                idx_v[n_chunks + r, pl.ds(k * _LANES, _LANES)] = zero16
        w0a = w_v[0, pl.ds(0, _LANES)]
        w0b = w_v[0, pl.ds(_LANES, _LANES)]
        w1a = w_v[1, pl.ds(0, _LANES)]
        w1b = w_v[1, pl.ds(_LANES, _LANES)]

        for c0 in range(_LEAD):
            pltpu.async_copy(t2_hbm.at[idx_v.at[c0]], rin.at[c0], gsem)

        def chunk_body(c, carry):
            bi = lax.bitwise_and(c, 1)
            bn = lax.bitwise_and(c + _LEAD, 1)
            bw = lax.bitwise_and(c, 1)
            pltpu.async_copy(t2_hbm.at[idx_v.at[c + _LEAD]], rin.at[bn], gsem)
            pltpu.make_async_copy(
                t2_hbm.at[idx_v.at[c]], rin.at[bi], gsem).wait()
            @pl.when(c >= 2)
            def _():
                pltpu.make_async_copy(
                    rout.at[bw],
                    out_hbm.at[pl.ds((base + (c - 2) * CHUNK) * D // 128,
                                     CHUNK * D // 128)],
                    osem).wait()

            for g in range(CHUNK // _LANES):
                lv16 = ln_v[c, pl.ds(g * _LANES, _LANES)]
                av16 = la_v[c, pl.ds(g * _LANES, _LANES)]
                for j in range(_LANES):
                    e = g * _LANES + j
                    lvj = _lane_bcast(lv16, j)
                    avj = _lane_bcast(av16, j)
                    r0 = rin[bi, e, pl.ds(0, _LANES)]
                    r1 = rin[bi, e, pl.ds(_LANES, _LANES)]
                    x0 = r0 + lvj * w0a + avj * w1a
                    x1 = r1 + lvj * w0b + avj * w1b
                    rout[bw, e // 4, pl.ds((e % 4) * D, _LANES)] = x0
                    rout[bw, e // 4, pl.ds((e % 4) * D + _LANES, _LANES)] = x1
            pltpu.async_copy(
                rout.at[bw],
                out_hbm.at[pl.ds((base + c * CHUNK) * D // 128,
                                 CHUNK * D // 128)],
                osem)
            return carry

        lax.fori_loop(0, n_chunks, chunk_body, 0)

        for c in (n_chunks - 2, n_chunks - 1):
            pltpu.make_async_copy(
                rout.at[c % 2],
                out_hbm.at[pl.ds((base + c * CHUNK) * D // 128,
                                 CHUNK * D // 128)],
                osem).wait()
        for r in range(_LEAD):
            pltpu.make_async_copy(
                t2_hbm.at[idx_v.at[n_chunks + r]],
                rin.at[(n_chunks + r) % 2], gsem).wait()

    x = sc_k(t2, gid, ln, la, w01)
    x3 = x.reshape(B, L * D // 128, 128)
    out5 = pl.pallas_call(
        _tanh_body,
        grid=(8,),
        in_specs=[pl.BlockSpec((128, L * D // 128, 128), lambda j: (j, 0, 0))],
        out_specs=pl.BlockSpec((L, 4, 1, 8, 128), lambda j: (0, 0, j, 0, 0)),
        out_shape=jax.ShapeDtypeStruct((L, 4, 8, 8, 128), jnp.float32),
    )(x3)
    return (out5.transpose(2, 4, 0, 1, 3).reshape(B, L, D))


# speedup vs baseline: 1.0009x; 1.0009x over previous
import functools

import jax
import jax.numpy as jnp
from jax import lax
from jax.experimental import pallas as pl
from jax.experimental.pallas import tpu as pltpu
from jax.experimental.pallas import tpu_sc as plsc

_LANES = 16
_LEAD = 1


def _t2_body(emb_ref, w2_ref, b_ref, out_ref):
    out_ref[...] = (
        jnp.dot(emb_ref[...], w2_ref[...], preferred_element_type=jnp.float32)
        + b_ref[...]
    )


def _lane_bcast(v, j):
    idx = jnp.full((_LANES,), j, dtype=jnp.int32)
    return jnp.take_along_axis(v, idx, axis=0, mode="promise_in_bounds")


def _tanh_body(x_ref, out_ref):
    n_grp = x_ref.shape[1]
    for i in range(n_grp):
        t = jnp.tanh(x_ref[:, i, :])
        tt = jnp.transpose(t)
        for q in range(4):
            st = tt[32 * q:32 * (q + 1), :]
            out_ref[4 * i + q, :, 0, :, :] = st.reshape(4, 8, 128)


def kernel(lngs, lats, grid_id, emb_table, W, b):
    B, L = lngs.shape
    V, D = emb_table.shape
    N = B * L

    t2 = pl.pallas_call(
        _t2_body,
        out_shape=jax.ShapeDtypeStruct((V, D), jnp.float32),
    )(emb_table, W[2:], b.reshape(1, D))

    info = plsc.get_sparse_core_info()
    NW = info.num_cores * info.num_subcores
    CHUNK = 128
    per_w = N // NW
    n_chunks = per_w // CHUNK
    assert per_w % CHUNK == 0 and N % NW == 0

    gid = grid_id.reshape(NW, n_chunks, CHUNK).astype(jnp.int32)
    ln = lngs.reshape(NW, n_chunks, CHUNK)
    la = lats.reshape(NW, n_chunks, CHUNK)
    w01 = W[:2]

    mesh = plsc.VectorSubcoreMesh(core_axis_name="c", subcore_axis_name="s")

    @functools.partial(
        pl.kernel,
        out_type=jax.ShapeDtypeStruct((N * D // 128, 128), jnp.float32),
        mesh=mesh,
        compiler_params=pltpu.CompilerParams(use_tc_tiling_on_sc=False),
        scratch_types=[
            pltpu.VMEM((n_chunks + _LEAD, CHUNK), jnp.int32),
            pltpu.VMEM((n_chunks, CHUNK), jnp.float32),
            pltpu.VMEM((n_chunks, CHUNK), jnp.float32),
            pltpu.VMEM((2, CHUNK, D), jnp.float32),
            pltpu.VMEM((2, CHUNK * D // 128, 128), jnp.float32),
            pltpu.VMEM((2, D), jnp.float32),
            pltpu.SemaphoreType.DMA,
            pltpu.SemaphoreType.DMA,
        ],
    )
    def sc_k(t2_hbm, gid_hbm, ln_hbm, la_hbm, w01_hbm, out_hbm,
             idx_v, ln_v, la_v, rin, rout, w_v, gsem, osem):
        wid = lax.axis_index("s") * info.num_cores + lax.axis_index("c")
        base = wid * per_w
        pltpu.sync_copy(gid_hbm.at[wid], idx_v.at[pl.ds(0, n_chunks)])
        pltpu.sync_copy(ln_hbm.at[wid], ln_v)
        pltpu.sync_copy(la_hbm.at[wid], la_v)
        pltpu.sync_copy(w01_hbm, w_v)
        zero16 = jnp.zeros((_LANES,), jnp.int32)
        for r in range(_LEAD):
            for k in range(CHUNK // _LANES):
                idx_v[n_chunks + r, pl.ds(k * _LANES, _LANES)] = zero16
        w0a = w_v[0, pl.ds(0, _LANES)]
        w0b = w_v[0, pl.ds(_LANES, _LANES)]
        w1a = w_v[1, pl.ds(0, _LANES)]
        w1b = w_v[1, pl.ds(_LANES, _LANES)]

        for c0 in range(_LEAD):
            pltpu.async_copy(t2_hbm.at[idx_v.at[c0]], rin.at[c0], gsem)

        def chunk_body(c, carry):
            bi = lax.bitwise_and(c, 1)
            bn = lax.bitwise_and(c + _LEAD, 1)
            bw = lax.bitwise_and(c, 1)
            pltpu.async_copy(t2_hbm.at[idx_v.at[c + _LEAD]], rin.at[bn], gsem)
            pltpu.make_async_copy(
                t2_hbm.at[idx_v.at[c]], rin.at[bi], gsem).wait()
            @pl.when(c >= 2)
            def _():
                pltpu.make_async_copy(
                    rout.at[bw],
                    out_hbm.at[pl.ds((base + (c - 2) * CHUNK) * D // 128,
                                     CHUNK * D // 128)],
                    osem).wait()

            for g in range(CHUNK // _LANES):
                lv16 = ln_v[c, pl.ds(g * _LANES, _LANES)]
                av16 = la_v[c, pl.ds(g * _LANES, _LANES)]
                for j in range(_LANES):
                    e = g * _LANES + j
                    lvj = _lane_bcast(lv16, j)
                    avj = _lane_bcast(av16, j)
                    r0 = rin[bi, e, pl.ds(0, _LANES)]
                    r1 = rin[bi, e, pl.ds(_LANES, _LANES)]
                    x0 = r0 + lvj * w0a + avj * w1a
                    x1 = r1 + lvj * w0b + avj * w1b
                    rout[bw, e // 4, pl.ds((e % 4) * D, _LANES)] = x0
                    rout[bw, e // 4, pl.ds((e % 4) * D + _LANES, _LANES)] = x1
            pltpu.async_copy(
                rout.at[bw],
                out_hbm.at[pl.ds((base + c * CHUNK) * D // 128,
                                 CHUNK * D // 128)],
                osem)
            return carry

        lax.fori_loop(0, n_chunks, chunk_body, 0)

        for c in (n_chunks - 2, n_chunks - 1):
            pltpu.make_async_copy(
                rout.at[c % 2],
                out_hbm.at[pl.ds((base + c * CHUNK) * D // 128,
                                 CHUNK * D // 128)],
                osem).wait()
        for r in range(_LEAD):
            pltpu.make_async_copy(
                t2_hbm.at[idx_v.at[n_chunks + r]],
                rin.at[(n_chunks + r) % 2], gsem).wait()

    x = sc_k(t2, gid, ln, la, w01)
    x3 = x.reshape(B, L * D // 128, 128)
    out5 = pl.pallas_call(
        _tanh_body,
        grid=(8,),
        in_specs=[pl.BlockSpec((128, L * D // 128, 128), lambda j: (j, 0, 0))],
        out_specs=pl.BlockSpec((L, 4, 1, 8, 128), lambda j: (0, 0, j, 0, 0)),
        out_shape=jax.ShapeDtypeStruct((L, 4, 8, 8, 128), jnp.float32),
    )(x3)
    return (out5.transpose(2, 4, 0, 1, 3).reshape(B, L, D))
